# baseline (speedup 1.0000x reference)
import functools

import jax
import jax.numpy as jnp
from jax import lax
from jax.experimental import pallas as pl
from jax.experimental.pallas import tpu as pltpu
from jax.experimental.pallas import tpu_sc as plsc

B = 4096
IN = 1024
VOCAB = 2000
EMB = 300
TOPK = 5

VOCABP = 2048
BM = 512
GM = 400
RB = 8
OROWS = RB * TOPK
NSTEPS = B // RB
NTILES = 32
STEPS_PER_TILE = NSTEPS // NTILES

_NEG = float("-inf")


def _head_body(x_ref, wg_ref, bg_ref, wf_ref, bf_ref,
               logits_ref, idx_ref, xw_ref):
    xb = x_ref[...]
    logits = (
        jnp.dot(xb, wg_ref[...], preferred_element_type=jnp.float32)
        + bg_ref[...]
    )
    logits_ref[...] = logits
    xw_ref[...] = (
        jnp.dot(xb, wf_ref[...], preferred_element_type=jnp.float32)
        + bf_ref[...]
    )
    col = lax.broadcasted_iota(jnp.int32, (BM, VOCAB), 1)
    l = logits
    cols = []
    for t in range(TOPK):
        m = jnp.max(l, axis=1, keepdims=True)
        a = jnp.min(jnp.where(l >= m, col, VOCAB), axis=1, keepdims=True)
        cols.append(a)
        if t + 1 < TOPK:
            l = jnp.where(col == a, _NEG, l)
    idx_ref[...] = jnp.concatenate(cols, axis=1)


def _table_body(we_ref, wm_ref, bm_ref, wf_ref, g_ref, k_scr):
    k_scr[...] = (
        jnp.dot(we_ref[...], wm_ref[...], preferred_element_type=jnp.float32)
        + bm_ref[...]
    )
    g_ref[...] = jnp.dot(k_scr[...], wf_ref[...],
                         preferred_element_type=jnp.float32)


GW = 16
NCHUNK = B * TOPK // GW


def _sc_gather_body(g_hbm, idx_hbm, out_hbm):
    def body(idx_v, o_v):
        pltpu.sync_copy(g_hbm.at[idx_v.at[0, 0]], o_v)

    pltpu.emit_pipeline(
        body,
        grid=(NCHUNK,),
        in_specs=[pl.BlockSpec((1, 1, GW), lambda i: (i, 0, 0))],
        out_specs=[pl.BlockSpec((GW, VOCABP), lambda i: (i, 0))],
        core_axis_name=("c", "s"),
        dimension_semantics=(pltpu.PARALLEL,),
    )(idx_hbm, out_hbm)


@functools.cache
def _sc_gather():
    mesh = plsc.VectorSubcoreMesh(core_axis_name="c", subcore_axis_name="s")
    return pl.kernel(
        _sc_gather_body,
        out_type=jax.ShapeDtypeStruct((B * TOPK, VOCABP), jnp.float32),
        mesh=mesh,
    )


def _tc_head(x, wg_t, bg, wf_t, bf):
    return pl.pallas_call(
        _head_body,
        grid=(B // BM,),
        in_specs=[
            pl.BlockSpec((BM, IN), lambda i: (i, 0)),
            pl.BlockSpec((IN, VOCAB), lambda i: (0, 0)),
            pl.BlockSpec((1, VOCAB), lambda i: (0, 0)),
            pl.BlockSpec((IN, VOCABP), lambda i: (0, 0)),
            pl.BlockSpec((1, VOCABP), lambda i: (0, 0)),
        ],
        out_specs=[
            pl.BlockSpec((BM, VOCAB), lambda i: (i, 0)),
            pl.BlockSpec((BM, TOPK), lambda i: (i, 0)),
            pl.BlockSpec((BM, VOCABP), lambda i: (i, 0)),
        ],
        out_shape=[
            jax.ShapeDtypeStruct((B, VOCAB), jnp.float32),
            jax.ShapeDtypeStruct((B, TOPK), jnp.int32),
            jax.ShapeDtypeStruct((B, VOCABP), jnp.float32),
        ],
        compiler_params=pltpu.CompilerParams(
            dimension_semantics=("arbitrary",),
        ),
    )(x, wg_t, bg, wf_t, bf)


def _tc_table(we, wm_t, bm, wf_t):
    return pl.pallas_call(
        _table_body,
        grid=(VOCAB // GM,),
        in_specs=[
            pl.BlockSpec((GM, EMB), lambda i: (i, 0)),
            pl.BlockSpec((EMB, IN), lambda i: (0, 0)),
            pl.BlockSpec((1, IN), lambda i: (0, 0)),
            pl.BlockSpec((IN, VOCABP), lambda i: (0, 0)),
        ],
        out_specs=pl.BlockSpec((GM, VOCABP), lambda i: (i, 0)),
        out_shape=jax.ShapeDtypeStruct((VOCAB, VOCABP), jnp.float32),
        scratch_shapes=[pltpu.VMEM((GM, IN), jnp.float32)],
        compiler_params=pltpu.CompilerParams(
            dimension_semantics=("arbitrary",),
        ),
    )(we, wm_t, bm, wf_t)


def kernel(x, word_embedding, W_gloss, b_gloss, W_mapper, b_mapper,
           W_fused, b_fused):
    wg_t = W_gloss.T
    wf_t = jnp.pad(W_fused.T, ((0, 0), (0, VOCABP - VOCAB)))
    wm_t = W_mapper.T
    bg = b_gloss.reshape(1, VOCAB)
    bf = jnp.pad(b_fused, (0, VOCABP - VOCAB)).reshape(1, VOCABP)
    bm = b_mapper.reshape(1, IN)

    logits, idx, xw = _tc_head(x, wg_t, bg, wf_t, bf)
    g = _tc_table(word_embedding, wm_t, bm, wf_t)
    gathered = _sc_gather()(g, idx.reshape(NCHUNK, 1, GW))
    fused = (
        gathered.reshape(B, TOPK, VOCABP)[:, :, :VOCAB]
        + xw[:, None, :VOCAB]
    )
    return (logits, fused, idx.reshape(-1))


# speedup vs baseline: 1.2003x; 1.2003x over previous
import functools

import jax
import jax.numpy as jnp
from jax import lax
from jax.experimental import pallas as pl
from jax.experimental.pallas import tpu as pltpu
from jax.experimental.pallas import tpu_sc as plsc

B = 4096
IN = 1024
VOCAB = 2000
EMB = 300
TOPK = 5

VOCABP = 2048
BM = 512
GM = 400
RB = 8
OROWS = RB * TOPK
NSTEPS = B // RB
NTILES = 32
STEPS_PER_TILE = NSTEPS // NTILES

_NEG = float("-inf")


def _head_body(x_ref, wg_ref, bg_ref, wf_ref, bf_ref,
               logits_ref, idx_ref, xw_ref):
    xb = x_ref[...]
    logits = (
        jnp.dot(xb, wg_ref[...], preferred_element_type=jnp.float32)
        + bg_ref[...]
    )
    logits_ref[...] = logits
    xw_ref[...] = (
        jnp.dot(xb, wf_ref[...], preferred_element_type=jnp.float32)
        + bf_ref[...]
    )
    col = lax.broadcasted_iota(jnp.int32, (BM, VOCAB), 1)
    l = logits
    cols = []
    for t in range(TOPK):
        m = jnp.max(l, axis=1, keepdims=True)
        a = jnp.min(jnp.where(l >= m, col, VOCAB), axis=1, keepdims=True)
        cols.append(a)
        if t + 1 < TOPK:
            l = jnp.where(col == a, _NEG, l)
    idx_ref[...] = jnp.concatenate(cols, axis=1)


def _table_body(we_ref, wm_ref, bm_ref, wf_ref, g_ref, k_scr):
    k_scr[...] = (
        jnp.dot(we_ref[...], wm_ref[...], preferred_element_type=jnp.float32)
        + bm_ref[...]
    )
    g_ref[...] = jnp.dot(k_scr[...], wf_ref[...],
                         preferred_element_type=jnp.float32)


def _sc_fuse_body(g_hbm, xw_hbm, idx_hbm, iot_hbm, out_hbm, xbuf):
    def body(idx_v, iot_v, o_v):
        pltpu.sync_copy(g_hbm.at[idx_v.at[0, 0]], o_v.at[0])
        pltpu.sync_copy(xw_hbm.at[iot_v.at[0, 0]], xbuf)

        @pl.loop(0, VOCAB // 16)
        def _(c):
            v = xbuf[0, pl.ds(c * 16, 16)]
            for t in range(TOPK):
                plsc.addupdate(o_v.at[0, t, pl.ds(c * 16, 16)], v)

    pltpu.emit_pipeline(
        body,
        grid=(B,),
        in_specs=[
            pl.BlockSpec((1, 1, TOPK), lambda i: (i, 0, 0)),
            pl.BlockSpec((1, 1, 1), lambda i: (i, 0, 0)),
        ],
        out_specs=[pl.BlockSpec((1, TOPK, VOCABP), lambda i: (i, 0, 0))],
        core_axis_name=("c", "s"),
        dimension_semantics=(pltpu.PARALLEL,),
    )(idx_hbm, iot_hbm, out_hbm)


@functools.cache
def _sc_fuse():
    mesh = plsc.VectorSubcoreMesh(core_axis_name="c", subcore_axis_name="s")
    return pl.kernel(
        _sc_fuse_body,
        out_type=jax.ShapeDtypeStruct((B, TOPK, VOCABP), jnp.float32),
        mesh=mesh,
        scratch_types=[pltpu.VMEM((1, VOCABP), jnp.float32)],
    )


def _tc_head(x, wg_t, bg, wf_t, bf):
    return pl.pallas_call(
        _head_body,
        grid=(B // BM,),
        in_specs=[
            pl.BlockSpec((BM, IN), lambda i: (i, 0)),
            pl.BlockSpec((IN, VOCAB), lambda i: (0, 0)),
            pl.BlockSpec((1, VOCAB), lambda i: (0, 0)),
            pl.BlockSpec((IN, VOCABP), lambda i: (0, 0)),
            pl.BlockSpec((1, VOCABP), lambda i: (0, 0)),
        ],
        out_specs=[
            pl.BlockSpec((BM, VOCAB), lambda i: (i, 0)),
            pl.BlockSpec((BM, TOPK), lambda i: (i, 0)),
            pl.BlockSpec((BM, VOCABP), lambda i: (i, 0)),
        ],
        out_shape=[
            jax.ShapeDtypeStruct((B, VOCAB), jnp.float32),
            jax.ShapeDtypeStruct((B, TOPK), jnp.int32),
            jax.ShapeDtypeStruct((B, VOCABP), jnp.float32),
        ],
        compiler_params=pltpu.CompilerParams(
            dimension_semantics=("arbitrary",),
        ),
    )(x, wg_t, bg, wf_t, bf)


def _tc_table(we, wm_t, bm, wf_t):
    return pl.pallas_call(
        _table_body,
        grid=(VOCAB // GM,),
        in_specs=[
            pl.BlockSpec((GM, EMB), lambda i: (i, 0)),
            pl.BlockSpec((EMB, IN), lambda i: (0, 0)),
            pl.BlockSpec((1, IN), lambda i: (0, 0)),
            pl.BlockSpec((IN, VOCABP), lambda i: (0, 0)),
        ],
        out_specs=pl.BlockSpec((GM, VOCABP), lambda i: (i, 0)),
        out_shape=jax.ShapeDtypeStruct((VOCAB, VOCABP), jnp.float32),
        scratch_shapes=[pltpu.VMEM((GM, IN), jnp.float32)],
        compiler_params=pltpu.CompilerParams(
            dimension_semantics=("arbitrary",),
        ),
    )(we, wm_t, bm, wf_t)


def kernel(x, word_embedding, W_gloss, b_gloss, W_mapper, b_mapper,
           W_fused, b_fused):
    wg_t = W_gloss.T
    wf_t = jnp.pad(W_fused.T, ((0, 0), (0, VOCABP - VOCAB)))
    wm_t = W_mapper.T
    bg = b_gloss.reshape(1, VOCAB)
    bf = jnp.pad(b_fused, (0, VOCABP - VOCAB)).reshape(1, VOCABP)
    bm = b_mapper.reshape(1, IN)

    logits, idx, xw = _tc_head(x, wg_t, bg, wf_t, bf)
    g = _tc_table(word_embedding, wm_t, bm, wf_t)
    iot = jnp.arange(B, dtype=jnp.int32).reshape(B, 1, 1)
    out3 = _sc_fuse()(g, xw, idx.reshape(B, 1, TOPK), iot)
    return (logits, out3[:, :, :VOCAB], idx.reshape(-1))


# speedup vs baseline: 1.4682x; 1.2232x over previous
import functools

import jax
import jax.numpy as jnp
from jax import lax
from jax.experimental import pallas as pl
from jax.experimental.pallas import tpu as pltpu
from jax.experimental.pallas import tpu_sc as plsc

B = 4096
IN = 1024
VOCAB = 2000
EMB = 300
TOPK = 5

VOCABP = 2048
BM = 512
GM = 400
RB = 8
OROWS = RB * TOPK
NSTEPS = B // RB
NTILES = 32
STEPS_PER_TILE = NSTEPS // NTILES

_NEG = float("-inf")


def _head_body(x_ref, wg_ref, bg_ref, wf_ref, bf_ref,
               logits_ref, idx_ref, xw_ref):
    xb = x_ref[...]
    logits = (
        jnp.dot(xb, wg_ref[...], preferred_element_type=jnp.float32)
        + bg_ref[...]
    )
    logits_ref[...] = logits
    xw_ref[...] = (
        jnp.dot(xb, wf_ref[...], preferred_element_type=jnp.float32)
        + bf_ref[...]
    )
    col = lax.broadcasted_iota(jnp.int32, (BM, VOCAB), 1)
    l = logits
    cols = []
    for t in range(TOPK):
        m = jnp.max(l, axis=1, keepdims=True)
        a = jnp.min(jnp.where(l >= m, col, VOCAB), axis=1, keepdims=True)
        cols.append(a)
        if t + 1 < TOPK:
            l = jnp.where(col == a, _NEG, l)
    idx_ref[...] = jnp.concatenate(cols, axis=1)


def _table_body(we_ref, wm_ref, bm_ref, wf_ref, g_ref, k_scr):
    k_scr[...] = (
        jnp.dot(we_ref[...], wm_ref[...], preferred_element_type=jnp.float32)
        + bm_ref[...]
    )
    g_ref[...] = jnp.dot(k_scr[...], wf_ref[...],
                         preferred_element_type=jnp.float32)


def _sc_fuse_body(g_hbm, idx_hbm, oidx_hbm, xw_hbm, out_hbm,
                  ibuf, obuf_i, xbuf, gbuf):
    wid = lax.axis_index("s") * 2 + lax.axis_index("c")

    @pl.loop(0, STEPS_PER_TILE)
    def _(s):
        step = wid * STEPS_PER_TILE + s
        pltpu.sync_copy(idx_hbm.at[step], ibuf)
        pltpu.sync_copy(oidx_hbm.at[step], obuf_i)
        pltpu.sync_copy(xw_hbm.at[pl.ds(step * RB, RB)], xbuf)
        pltpu.sync_copy(g_hbm.at[ibuf.at[0]], gbuf)

        @pl.loop(0, VOCAB // 16)
        def _(c):
            for br in range(RB):
                v = xbuf[br, pl.ds(c * 16, 16)]
                for t in range(TOPK):
                    plsc.addupdate(gbuf.at[TOPK * br + t, pl.ds(c * 16, 16)], v)

        pltpu.sync_copy(gbuf, out_hbm.at[obuf_i.at[0]])


@functools.cache
def _sc_fuse():
    mesh = plsc.VectorSubcoreMesh(core_axis_name="c", subcore_axis_name="s")
    return pl.kernel(
        _sc_fuse_body,
        out_type=jax.ShapeDtypeStruct((B * 8, VOCABP), jnp.float32),
        mesh=mesh,
        scratch_types=[
            pltpu.VMEM((1, OROWS), jnp.int32),
            pltpu.VMEM((1, OROWS), jnp.int32),
            pltpu.VMEM((RB, VOCABP), jnp.float32),
            pltpu.VMEM((OROWS, VOCABP), jnp.float32),
        ],
    )


def _tc_head(x, wg_t, bg, wf_t, bf):
    return pl.pallas_call(
        _head_body,
        grid=(B // BM,),
        in_specs=[
            pl.BlockSpec((BM, IN), lambda i: (i, 0)),
            pl.BlockSpec((IN, VOCAB), lambda i: (0, 0)),
            pl.BlockSpec((1, VOCAB), lambda i: (0, 0)),
            pl.BlockSpec((IN, VOCABP), lambda i: (0, 0)),
            pl.BlockSpec((1, VOCABP), lambda i: (0, 0)),
        ],
        out_specs=[
            pl.BlockSpec((BM, VOCAB), lambda i: (i, 0)),
            pl.BlockSpec((BM, TOPK), lambda i: (i, 0)),
            pl.BlockSpec((BM, VOCABP), lambda i: (i, 0)),
        ],
        out_shape=[
            jax.ShapeDtypeStruct((B, VOCAB), jnp.float32),
            jax.ShapeDtypeStruct((B, TOPK), jnp.int32),
            jax.ShapeDtypeStruct((B, VOCABP), jnp.float32),
        ],
        compiler_params=pltpu.CompilerParams(
            dimension_semantics=("arbitrary",),
        ),
    )(x, wg_t, bg, wf_t, bf)


def _tc_table(we, wm_t, bm, wf_t):
    return pl.pallas_call(
        _table_body,
        grid=(VOCAB // GM,),
        in_specs=[
            pl.BlockSpec((GM, EMB), lambda i: (i, 0)),
            pl.BlockSpec((EMB, IN), lambda i: (0, 0)),
            pl.BlockSpec((1, IN), lambda i: (0, 0)),
            pl.BlockSpec((IN, VOCABP), lambda i: (0, 0)),
        ],
        out_specs=pl.BlockSpec((GM, VOCABP), lambda i: (i, 0)),
        out_shape=jax.ShapeDtypeStruct((VOCAB, VOCABP), jnp.float32),
        scratch_shapes=[pltpu.VMEM((GM, IN), jnp.float32)],
        compiler_params=pltpu.CompilerParams(
            dimension_semantics=("arbitrary",),
        ),
    )(we, wm_t, bm, wf_t)


def kernel(x, word_embedding, W_gloss, b_gloss, W_mapper, b_mapper,
           W_fused, b_fused):
    wg_t = W_gloss.T
    wf_t = jnp.pad(W_fused.T, ((0, 0), (0, VOCABP - VOCAB)))
    wm_t = W_mapper.T
    bg = b_gloss.reshape(1, VOCAB)
    bf = jnp.pad(b_fused, (0, VOCABP - VOCAB)).reshape(1, VOCABP)
    bm = b_mapper.reshape(1, IN)

    logits, idx, xw = _tc_head(x, wg_t, bg, wf_t, bf)
    g = _tc_table(word_embedding, wm_t, bm, wf_t)
    r = jnp.arange(B * TOPK, dtype=jnp.int32)
    oidx = (8 * (r // TOPK) + r % TOPK).reshape(NSTEPS, 1, OROWS)
    out2 = _sc_fuse()(g, idx.reshape(NSTEPS, 1, OROWS), oidx, xw)
    fused = out2.reshape(B, 8, VOCABP)[:, :TOPK, :VOCAB]
    return (logits, fused, idx.reshape(-1))


# speedup vs baseline: 1.4919x; 1.0161x over previous
import functools

import jax
import jax.numpy as jnp
from jax import lax
from jax.experimental import pallas as pl
from jax.experimental.pallas import tpu as pltpu
from jax.experimental.pallas import tpu_sc as plsc

B = 4096
IN = 1024
VOCAB = 2000
EMB = 300
TOPK = 5

VOCABP = 2048
BM = 512
GM = 400
RB = 8
OROWS = RB * TOPK
NSTEPS = B // RB
NTILES = 32
STEPS_PER_TILE = NSTEPS // NTILES

_NEG = float("-inf")


def _head_body(x_ref, wg_ref, bg_ref, wf_ref, bf_ref,
               logits_ref, idx_ref, xw_ref):
    xb = x_ref[...]
    logits = (
        jnp.dot(xb, wg_ref[...], preferred_element_type=jnp.float32)
        + bg_ref[...]
    )
    logits_ref[...] = logits
    xw_ref[...] = (
        jnp.dot(xb, wf_ref[...], preferred_element_type=jnp.float32)
        + bf_ref[...]
    )
    col = lax.broadcasted_iota(jnp.int32, (BM, VOCAB), 1)
    l = logits
    cols = []
    for t in range(TOPK):
        m = jnp.max(l, axis=1, keepdims=True)
        a = jnp.min(jnp.where(l >= m, col, VOCAB), axis=1, keepdims=True)
        cols.append(a)
        if t + 1 < TOPK:
            l = jnp.where(col == a, _NEG, l)
    idx_ref[...] = jnp.concatenate(cols, axis=1)


def _table_body(we_ref, wm_ref, bm_ref, wf_ref, g_ref, k_scr):
    k_scr[...] = (
        jnp.dot(we_ref[...], wm_ref[...], preferred_element_type=jnp.float32)
        + bm_ref[...]
    )
    g_ref[...] = jnp.dot(k_scr[...], wf_ref[...],
                         preferred_element_type=jnp.float32)


def _sc_fuse_body(g_hbm, idx_hbm, oidx_hbm, xw_hbm, out_hbm,
                  ibuf, obuf_i, xbuf, gbuf):
    wid = lax.axis_index("s") * 2 + lax.axis_index("c")

    @pl.loop(0, STEPS_PER_TILE)
    def _(s):
        step = wid * STEPS_PER_TILE + s
        pltpu.sync_copy(idx_hbm.at[step], ibuf)
        pltpu.sync_copy(oidx_hbm.at[step], obuf_i)
        pltpu.sync_copy(xw_hbm.at[pl.ds(step * RB, RB)], xbuf)
        pltpu.sync_copy(g_hbm.at[ibuf.at[0]], gbuf)

        @pl.loop(0, VOCAB // 16, unroll=5)
        def _(c):
            for br in range(RB):
                v = xbuf[br, pl.ds(c * 16, 16)]
                for t in range(TOPK):
                    plsc.addupdate(gbuf.at[TOPK * br + t, pl.ds(c * 16, 16)], v)

        pltpu.sync_copy(gbuf, out_hbm.at[obuf_i.at[0]])


@functools.cache
def _sc_fuse():
    mesh = plsc.VectorSubcoreMesh(core_axis_name="c", subcore_axis_name="s")
    return pl.kernel(
        _sc_fuse_body,
        out_type=jax.ShapeDtypeStruct((B * 8, VOCABP), jnp.float32),
        mesh=mesh,
        scratch_types=[
            pltpu.VMEM((1, OROWS), jnp.int32),
            pltpu.VMEM((1, OROWS), jnp.int32),
            pltpu.VMEM((RB, VOCABP), jnp.float32),
            pltpu.VMEM((OROWS, VOCABP), jnp.float32),
        ],
    )


def _tc_head(x, wg_t, bg, wf_t, bf):
    return pl.pallas_call(
        _head_body,
        grid=(B // BM,),
        in_specs=[
            pl.BlockSpec((BM, IN), lambda i: (i, 0)),
            pl.BlockSpec((IN, VOCAB), lambda i: (0, 0)),
            pl.BlockSpec((1, VOCAB), lambda i: (0, 0)),
            pl.BlockSpec((IN, VOCABP), lambda i: (0, 0)),
            pl.BlockSpec((1, VOCABP), lambda i: (0, 0)),
        ],
        out_specs=[
            pl.BlockSpec((BM, VOCAB), lambda i: (i, 0)),
            pl.BlockSpec((BM, TOPK), lambda i: (i, 0)),
            pl.BlockSpec((BM, VOCABP), lambda i: (i, 0)),
        ],
        out_shape=[
            jax.ShapeDtypeStruct((B, VOCAB), jnp.float32),
            jax.ShapeDtypeStruct((B, TOPK), jnp.int32),
            jax.ShapeDtypeStruct((B, VOCABP), jnp.float32),
        ],
        compiler_params=pltpu.CompilerParams(
            dimension_semantics=("arbitrary",),
        ),
    )(x, wg_t, bg, wf_t, bf)


def _tc_table(we, wm_t, bm, wf_t):
    return pl.pallas_call(
        _table_body,
        grid=(VOCAB // GM,),
        in_specs=[
            pl.BlockSpec((GM, EMB), lambda i: (i, 0)),
            pl.BlockSpec((EMB, IN), lambda i: (0, 0)),
            pl.BlockSpec((1, IN), lambda i: (0, 0)),
            pl.BlockSpec((IN, VOCABP), lambda i: (0, 0)),
        ],
        out_specs=pl.BlockSpec((GM, VOCABP), lambda i: (i, 0)),
        out_shape=jax.ShapeDtypeStruct((VOCAB, VOCABP), jnp.float32),
        scratch_shapes=[pltpu.VMEM((GM, IN), jnp.float32)],
        compiler_params=pltpu.CompilerParams(
            dimension_semantics=("arbitrary",),
        ),
    )(we, wm_t, bm, wf_t)


def kernel(x, word_embedding, W_gloss, b_gloss, W_mapper, b_mapper,
           W_fused, b_fused):
    wg_t = W_gloss.T
    wf_t = jnp.pad(W_fused.T, ((0, 0), (0, VOCABP - VOCAB)))
    wm_t = W_mapper.T
    bg = b_gloss.reshape(1, VOCAB)
    bf = jnp.pad(b_fused, (0, VOCABP - VOCAB)).reshape(1, VOCABP)
    bm = b_mapper.reshape(1, IN)

    logits, idx, xw = _tc_head(x, wg_t, bg, wf_t, bf)
    g = _tc_table(word_embedding, wm_t, bm, wf_t)
    r = jnp.arange(B * TOPK, dtype=jnp.int32)
    oidx = (8 * (r // TOPK) + r % TOPK).reshape(NSTEPS, 1, OROWS)
    out2 = _sc_fuse()(g, idx.reshape(NSTEPS, 1, OROWS), oidx, xw)
    fused = out2.reshape(B, 8, VOCABP)[:, :TOPK, :VOCAB]
    return (logits, fused, idx.reshape(-1))
